# baseline (speedup 1.0000x reference)
import jax
import jax.numpy as jnp
from jax.experimental import pallas as pl
from jax.experimental.pallas import tpu as pltpu

_IMGS = 4


def _relu6(x):
    return jnp.minimum(jnp.maximum(x, 0.0), 6.0)


def _block_kernel(x_ref, w1_ref, ws1_ref, wt_ref,
                  o12_ref, o3_ref,
                  s_y1, s_b2m, s_y2x, s_x):
    H = W = 56
    Ho = Wo = 28

    s_y1[...] = jnp.zeros_like(s_y1)
    s_b2m[...] = jnp.zeros_like(s_b2m)
    s_y2x[...] = jnp.zeros_like(s_y2x)
    s_x[...] = jnp.full(s_x.shape, -jnp.inf, s_x.dtype)
    for i in range(_IMGS):
        x = x_ref[i]
        y = jnp.dot(x.reshape(H * W, 64), w1_ref[...],
                    preferred_element_type=jnp.float32)
        y = _relu6(y)
        s_y1[i, 1:57, 1:57, :] = y[:, :128].reshape(H, W, 128)
        y2 = y[:, 128:].reshape(H, W, 64)
        s_y2x[i, 1:57, 1:56, 0:64] = y2[:, :55, :]
        s_y2x[i, 1:57, 0:56, 64:128] = y2
        s_y2x[i, 1:57, 0:55, 128:192] = y2[:, 1:, :]
        s_x[i, 1:57, 1:57, :] = x.astype(jnp.float32)

    for i in range(_IMGS):
        acc = jnp.zeros((H * W, 96), jnp.float32)
        for ky in range(3):
            tap = s_y2x[i, ky:ky + H, 0:W, :]
            acc = acc + jnp.dot(tap.reshape(H * W, 192), ws1_ref[ky],
                                preferred_element_type=jnp.float32)
        s_b2m[i, 1:57, 1:57, 0:96] = _relu6(acc).reshape(H, W, 96)

    for i in range(_IMGS):
        acc12 = jnp.zeros((Ho * Wo, 256), jnp.float32)
        m = None
        for ky in range(3):
            for kx in range(3):
                rows = pl.ds(ky, Ho, 2)
                cols = pl.ds(kx, Wo, 2)
                t12 = jnp.concatenate([s_y1[i, rows, cols, :],
                                       s_b2m[i, rows, cols, :]], axis=-1)
                acc12 = acc12 + jnp.dot(t12.reshape(Ho * Wo, 256),
                                        wt_ref[ky * 3 + kx],
                                        preferred_element_type=jnp.float32)
                t3 = s_x[i, rows, cols, :]
                m = t3 if m is None else jnp.maximum(m, t3)

        o12_ref[i] = _relu6(acc12).reshape(Ho, Wo, 256).astype(o12_ref.dtype)
        o3_ref[i] = m.astype(o3_ref.dtype)


@jax.jit
def kernel(x_nchw, w_1x1, w_b1c2, w_b2c2, w_b2c3):
    N, Cin, H, W = x_nchw.shape
    Ho, Wo = H // 2, W // 2
    x = jnp.transpose(x_nchw, (0, 2, 3, 1))

    ws1 = jnp.stack([
        jnp.concatenate([w_b2c2[3 * ky + 0], w_b2c2[3 * ky + 1],
                         w_b2c2[3 * ky + 2]], axis=0)
        for ky in range(3)])

    wt = jnp.zeros((9, 256, 256), jnp.float32)
    wt = wt.at[:, 0:128, 0:160].set(w_b1c2)
    wt = wt.at[:, 128:224, 160:256].set(w_b2c3)

    o12, o3 = pl.pallas_call(
        _block_kernel,
        out_shape=(jax.ShapeDtypeStruct((N, Ho, Wo, 256), jnp.bfloat16),
                   jax.ShapeDtypeStruct((N, Ho, Wo, Cin), jnp.bfloat16)),
        grid=(N // _IMGS,),
        in_specs=[pl.BlockSpec((_IMGS, H, W, Cin), lambda n: (n, 0, 0, 0)),
                  pl.BlockSpec((Cin, 192), lambda n: (0, 0)),
                  pl.BlockSpec((3, 192, 96), lambda n: (0, 0, 0)),
                  pl.BlockSpec((9, 256, 256), lambda n: (0, 0, 0))],
        out_specs=[pl.BlockSpec((_IMGS, Ho, Wo, 256), lambda n: (n, 0, 0, 0)),
                   pl.BlockSpec((_IMGS, Ho, Wo, Cin), lambda n: (n, 0, 0, 0))],
        scratch_shapes=[pltpu.VMEM((_IMGS, 58, 57, 128), jnp.float32),
                        pltpu.VMEM((_IMGS, 58, 57, 128), jnp.float32),
                        pltpu.VMEM((_IMGS, 58, 56, 192), jnp.float32),
                        pltpu.VMEM((_IMGS, 58, 57, 64), jnp.float32)],
        compiler_params=pltpu.CompilerParams(
            dimension_semantics=("parallel",),
            vmem_limit_bytes=100 * 1024 * 1024),
    )(x.astype(jnp.bfloat16), w_1x1.astype(jnp.bfloat16), ws1, wt)

    out = jnp.concatenate([o12, o3], axis=-1).astype(jnp.float32)
    return jnp.transpose(out, (0, 3, 1, 2))


# speedup vs baseline: 1.1932x; 1.1932x over previous
import jax
import jax.numpy as jnp
from jax.experimental import pallas as pl
from jax.experimental.pallas import tpu as pltpu

_IMGS = 4


def _relu6(x):
    return jnp.minimum(jnp.maximum(x, 0.0), 6.0)


def _block_kernel(x_ref, w1_ref, ws1_ref, wt_ref,
                  o12_ref, o3_ref,
                  s_y1, s_b2m, s_y2x, s_x):
    H = W = 56
    Ho = Wo = 28

    s_y1[:, 0:1] = jnp.zeros_like(s_y1[:, 0:1])
    s_y1[:, 57:58] = jnp.zeros_like(s_y1[:, 57:58])
    s_y1[:, :, 0:1] = jnp.zeros_like(s_y1[:, :, 0:1])
    s_b2m[:, 0:1] = jnp.zeros_like(s_b2m[:, 0:1])
    s_b2m[:, 57:58] = jnp.zeros_like(s_b2m[:, 57:58])
    s_b2m[:, :, 0:1] = jnp.zeros_like(s_b2m[:, :, 0:1])
    s_b2m[:, :, :, 96:128] = jnp.zeros_like(s_b2m[:, :, :, 96:128])
    s_y2x[:, 0:1] = jnp.zeros_like(s_y2x[:, 0:1])
    s_y2x[:, 57:58] = jnp.zeros_like(s_y2x[:, 57:58])
    s_y2x[:, :, 0:1, 0:64] = jnp.zeros_like(s_y2x[:, :, 0:1, 0:64])
    s_y2x[:, :, 55:56, 128:192] = jnp.zeros_like(s_y2x[:, :, 55:56, 128:192])
    ninf = jnp.full((), -jnp.inf, s_x.dtype)
    s_x[:, 0:1] = jnp.broadcast_to(ninf, s_x[:, 0:1].shape)
    s_x[:, 57:58] = jnp.broadcast_to(ninf, s_x[:, 57:58].shape)
    s_x[:, :, 0:1] = jnp.broadcast_to(ninf, s_x[:, :, 0:1].shape)
    for i in range(_IMGS):
        x = x_ref[i]
        y = jnp.dot(x.reshape(H * W, 64), w1_ref[...],
                    preferred_element_type=jnp.float32)
        y = _relu6(y)
        s_y1[i, 1:57, 1:57, :] = y[:, :128].reshape(H, W, 128)
        y2 = y[:, 128:].reshape(H, W, 64)
        s_y2x[i, 1:57, 1:56, 0:64] = y2[:, :55, :]
        s_y2x[i, 1:57, 0:56, 64:128] = y2
        s_y2x[i, 1:57, 0:55, 128:192] = y2[:, 1:, :]
        s_x[i, 1:57, 1:57, :] = x

    for i in range(_IMGS):
        acc = jnp.zeros((H * W, 96), jnp.float32)
        for ky in range(3):
            tap = s_y2x[i, ky:ky + H, 0:W, :]
            acc = acc + jnp.dot(tap.reshape(H * W, 192), ws1_ref[ky],
                                preferred_element_type=jnp.float32)
        s_b2m[i, 1:57, 1:57, 0:96] = _relu6(acc).reshape(H, W, 96)

    for i in range(_IMGS):
        acc12 = jnp.zeros((Ho * Wo, 256), jnp.float32)
        m = None
        for ky in range(3):
            for kx in range(3):
                rows = pl.ds(ky, Ho, 2)
                cols = pl.ds(kx, Wo, 2)
                t12 = jnp.concatenate([s_y1[i, rows, cols, :],
                                       s_b2m[i, rows, cols, :]], axis=-1)
                acc12 = acc12 + jnp.dot(t12.reshape(Ho * Wo, 256),
                                        wt_ref[ky * 3 + kx],
                                        preferred_element_type=jnp.float32)
                t3 = s_x[i, rows, cols, :]
                m = t3 if m is None else jnp.maximum(m, t3)

        o12_ref[i] = _relu6(acc12).reshape(Ho, Wo, 256).astype(o12_ref.dtype)
        o3_ref[i] = m.astype(o3_ref.dtype)


@jax.jit
def kernel(x_nchw, w_1x1, w_b1c2, w_b2c2, w_b2c3):
    N, Cin, H, W = x_nchw.shape
    Ho, Wo = H // 2, W // 2
    x = jnp.transpose(x_nchw, (0, 2, 3, 1))

    ws1 = jnp.stack([
        jnp.concatenate([w_b2c2[3 * ky + 0], w_b2c2[3 * ky + 1],
                         w_b2c2[3 * ky + 2]], axis=0)
        for ky in range(3)])

    wt = jnp.zeros((9, 256, 256), jnp.float32)
    wt = wt.at[:, 0:128, 0:160].set(w_b1c2)
    wt = wt.at[:, 128:224, 160:256].set(w_b2c3)

    o12, o3 = pl.pallas_call(
        _block_kernel,
        out_shape=(jax.ShapeDtypeStruct((N, Ho, Wo, 256), jnp.float32),
                   jax.ShapeDtypeStruct((N, Ho, Wo, Cin), jnp.float32)),
        grid=(N // _IMGS,),
        in_specs=[pl.BlockSpec((_IMGS, H, W, Cin), lambda n: (n, 0, 0, 0)),
                  pl.BlockSpec((Cin, 192), lambda n: (0, 0)),
                  pl.BlockSpec((3, 192, 96), lambda n: (0, 0, 0)),
                  pl.BlockSpec((9, 256, 256), lambda n: (0, 0, 0))],
        out_specs=[pl.BlockSpec((_IMGS, Ho, Wo, 256), lambda n: (n, 0, 0, 0)),
                   pl.BlockSpec((_IMGS, Ho, Wo, Cin), lambda n: (n, 0, 0, 0))],
        scratch_shapes=[pltpu.VMEM((_IMGS, 58, 57, 128), jnp.float32),
                        pltpu.VMEM((_IMGS, 58, 57, 128), jnp.float32),
                        pltpu.VMEM((_IMGS, 58, 56, 192), jnp.float32),
                        pltpu.VMEM((_IMGS, 58, 57, 64), jnp.float32)],
        compiler_params=pltpu.CompilerParams(
            dimension_semantics=("parallel",),
            vmem_limit_bytes=100 * 1024 * 1024),
    )(x, w_1x1, ws1, wt)

    out = jnp.concatenate([o12, o3], axis=-1)
    return jnp.transpose(out, (0, 3, 1, 2))
